# baseline (speedup 1.0000x reference)
import functools

import jax
import jax.numpy as jnp
from jax import lax
from jax.experimental import pallas as pl
from jax.experimental.pallas import tpu as pltpu
from jax.experimental.pallas import tpu_sc as plsc

VOCAB = 100000
DIM = 128
OFFSET = 30522
NC = 2
NS = 16
L = 16
NW = NC * NS
NENT = 50
NB = 4096
NTOK = NB * NENT
C = 128
NCHUNK = 50
NFULL = 6
TROW = NB // C
NBUF = 5
K = 3

_mesh = plsc.VectorSubcoreMesh(core_axis_name="c", subcore_axis_name="s")


@functools.partial(
    pl.kernel,
    mesh=_mesh,
    out_type=jax.ShapeDtypeStruct((NTOK, DIM), jnp.float32),
    scratch_types=[
        pltpu.VMEM((NCHUNK, C), jnp.int32),
        *[pltpu.VMEM((C, DIM), jnp.float32) for _ in range(NBUF)],
        pltpu.SemaphoreType.DMA,
        *[pltpu.SemaphoreType.DMA for _ in range(2 * NBUF)],
    ],
)
def _gather_kernel(ids_hbm, table_hbm, out_hbm, idx_v, *bufs_sems):
    bufs = bufs_sems[:NBUF]
    sem_i = bufs_sems[NBUF]
    sgs = bufs_sems[NBUF + 1:NBUF + 1 + NBUF]
    sss = bufs_sems[NBUF + 1 + NBUF:]
    wid = lax.axis_index("s") * NC + lax.axis_index("c")

    for j in range(NFULL):
        t = NFULL * wid + j
        lt = t // TROW
        bt = t - lt * TROW
        pltpu.async_copy(
            ids_hbm.at[pl.ds(pl.multiple_of(lt * 8, 8), 8),
                       pl.ds(pl.multiple_of(bt * C, C), C)],
            idx_v.at[pl.ds(j * 8, 8)],
            sem_i,
        )
    pltpu.async_copy(
        ids_hbm.at[pl.ds(NFULL * 8, 2),
                   pl.ds(pl.multiple_of(wid * C, C), C)],
        idx_v.at[pl.ds(NFULL * 8, 2)],
        sem_i,
    )
    pltpu.make_async_copy(ids_hbm.at[pl.ds(0, NCHUNK), pl.ds(0, C)],
                          idx_v, sem_i).wait()

    def _sub(c):
        for j in range(C // L):
            sl = (c, pl.ds(j * L, L))
            idx_v[sl] = idx_v[sl] - OFFSET

    def _out_row(c):
        if isinstance(c, int) and c >= NFULL * 8:
            return pl.multiple_of(c * NB + wid * C, 8)
        t = NFULL * wid + c // 8
        lt = t // TROW
        bt = t - lt * TROW
        return pl.multiple_of((lt * 8 + c % 8) * NB + bt * C, 8)

    def _fire_gather(c, p):
        pltpu.async_copy(table_hbm.at[idx_v.at[c]], bufs[p], sgs[p])

    def _fire_store(c, p):
        pltpu.async_copy(bufs[p], out_hbm.at[pl.ds(_out_row(c), C)], sss[p])

    def _drain(p, sem):
        pltpu.make_async_copy(table_hbm.at[pl.ds(0, C)], bufs[p], sem[p]).wait()

    for c in range(K):
        _sub(c)
        _fire_gather(c, c)

    def _step(c, p, q, drain_q=False, fire=True):
        _drain(p, sgs)
        _fire_store(c, p)
        if drain_q:
            _drain(q, sss)
        if fire:
            _sub(c + K)
            _fire_gather(c + K, q)

    _step(0, 0, K % NBUF)
    _step(1, 1, (1 + K) % NBUF)

    def _lap(k, carry):
        c0 = 2 + k * NBUF
        for b in range(NBUF):
            _step(c0 + b, (2 + b) % NBUF, b, drain_q=True)
        return carry

    lax.fori_loop(0, (NCHUNK - K - 2) // NBUF, _lap, 0)

    for c in range(NCHUNK - K, NCHUNK):
        _step(c, c % NBUF, (c - 2) % NBUF, drain_q=True, fire=False)
    _drain((NCHUNK - 2) % NBUF, sss)
    _drain((NCHUNK - 1) % NBUF, sss)


def kernel(entity_ids, entity_emb):
    bsz, num_ent = entity_ids.shape
    out = _gather_kernel(entity_ids.T, entity_emb)
    return out.reshape(num_ent, bsz, DIM).transpose(1, 0, 2)
